# Initial kernel scaffold; baseline (speedup 1.0000x reference)
import jax
import jax.numpy as jnp
from jax.experimental import pallas as pl


def kernel(x, edge_index, Wm, bm, W1, b1, W2, b2):
    raise NotImplementedError("write your pallas kernel here")


# speedup vs baseline: 23.9686x; 23.9686x over previous
import functools

import jax
import jax.numpy as jnp
from jax import lax
from jax.experimental import pallas as pl
from jax.experimental.pallas import tpu as pltpu
from jax.experimental.pallas import tpu_sc as plsc

N = 10000
N_PAD = 10240
E = 320000
NC = 2
NS = 16
EPT = E // NS
K = 80
CH = EPT // K
NPT = N_PAD // NS
ZR = 128

_mesh = lambda: plsc.VectorSubcoreMesh(core_axis_name="c", subcore_axis_name="s")
_sc_params = lambda: pltpu.CompilerParams(use_tc_tiling_on_sc=False)


def _zero_fill(ref, nrows, width):
    z = jnp.zeros((16,), jnp.float32)

    def row(r, _):
        def col(cc, __):
            ref[r, pl.ds(cc * 16, 16)] = z
            return 0

        lax.fori_loop(0, width // 16, col, 0)
        return 0

    lax.fori_loop(0, nrows, row, 0)


@functools.lru_cache(maxsize=None)
def _make_deg():
    @functools.partial(
        pl.kernel,
        out_type=jax.ShapeDtypeStruct((NC, N_PAD, 16), jnp.float32),
        mesh=_mesh(),
        compiler_params=_sc_params(),
        scratch_types=[
            pltpu.VMEM((CH, K), jnp.int32),
            pltpu.VMEM((K, 16), jnp.float32),
            pltpu.VMEM((ZR, 16), jnp.float32),
            pltpu.VMEM_SHARED((N_PAD, 16), jnp.float32),
        ],
    )
    def deg_kernel(dst_hbm, out_hbm, dstv, ones_v, zbuf, acc):
        c = lax.axis_index("c")
        s = lax.axis_index("s")

        _zero_fill(zbuf, ZR, 16)
        one = jnp.ones((16,), jnp.float32)

        def fill_ones(r, _):
            ones_v[r, :] = one
            return 0

        lax.fori_loop(0, K, fill_ones, 0)
        for t in range(NPT // ZR):
            pltpu.sync_copy(zbuf, acc.at[pl.ds(s * NPT + t * ZR, ZR)])
        pltpu.sync_copy(dst_hbm.at[s], dstv)
        plsc.subcore_barrier()

        def chunk(j, _):
            pltpu.sync_copy(ones_v, acc.at[dstv.at[j]], add=True)
            return 0

        lax.fori_loop(0, CH, chunk, 0)
        plsc.subcore_barrier()
        pltpu.sync_copy(acc.at[pl.ds(s * NPT, NPT)], out_hbm.at[c, pl.ds(s * NPT, NPT)])

    return deg_kernel


@functools.lru_cache(maxsize=None)
def _make_prop(DH):
    @functools.partial(
        pl.kernel,
        out_type=jax.ShapeDtypeStruct((NC, N_PAD, DH), jnp.float32),
        mesh=_mesh(),
        compiler_params=_sc_params(),
        scratch_types=[
            pltpu.VMEM((CH, K), jnp.int32),
            pltpu.VMEM((CH, K), jnp.int32),
            pltpu.VMEM((K, DH), jnp.float32),
            pltpu.VMEM((K, DH), jnp.float32),
            pltpu.VMEM((ZR, DH), jnp.float32),
            pltpu.VMEM_SHARED((N_PAD, DH), jnp.float32),
            pltpu.SemaphoreType.DMA,
            pltpu.SemaphoreType.DMA,
        ],
    )
    def prop_kernel(tabA_hbm, tabB_hbm, src_hbm, dst_hbm, out_hbm,
                    srcv, dstv, rows0, rows1, zbuf, acc, sem0, sem1):
        c = lax.axis_index("c")
        s = lax.axis_index("s")

        _zero_fill(zbuf, ZR, DH)
        for t in range(NPT // ZR):
            pltpu.sync_copy(zbuf, acc.at[pl.ds(s * NPT + t * ZR, ZR)])
        pltpu.sync_copy(src_hbm.at[s], srcv)
        pltpu.sync_copy(dst_hbm.at[s], dstv)
        plsc.subcore_barrier()

        def edge_loop(tab_hbm):
            pltpu.async_copy(tab_hbm.at[srcv.at[0]], rows0, sem0)

            def pair(jj, _):
                a = jj * 2
                pltpu.async_copy(tab_hbm.at[srcv.at[a + 1]], rows1, sem1)
                pltpu.make_async_copy(tab_hbm.at[srcv.at[a]], rows0, sem0).wait()
                pltpu.sync_copy(rows0, acc.at[dstv.at[a]], add=True)
                pltpu.async_copy(tab_hbm.at[srcv.at[a + 2]], rows0, sem0)
                pltpu.make_async_copy(tab_hbm.at[srcv.at[a + 1]], rows1, sem1).wait()
                pltpu.sync_copy(rows1, acc.at[dstv.at[a + 1]], add=True)
                return 0

            lax.fori_loop(0, (CH - 2) // 2, pair, 0)
            pltpu.async_copy(tab_hbm.at[srcv.at[CH - 1]], rows1, sem1)
            pltpu.make_async_copy(tab_hbm.at[srcv.at[CH - 2]], rows0, sem0).wait()
            pltpu.sync_copy(rows0, acc.at[dstv.at[CH - 2]], add=True)
            pltpu.make_async_copy(tab_hbm.at[srcv.at[CH - 1]], rows1, sem1).wait()
            pltpu.sync_copy(rows1, acc.at[dstv.at[CH - 1]], add=True)

        @pl.when(c == 0)
        def _():
            edge_loop(tabA_hbm)

        @pl.when(c == 1)
        def _():
            edge_loop(tabB_hbm)

        plsc.subcore_barrier()
        pltpu.sync_copy(acc.at[pl.ds(s * NPT, NPT)], out_hbm.at[c, pl.ds(s * NPT, NPT)])

    return prop_kernel


BLK = 2000


def _mlp_body(x_ref, wm_ref, bm_ref, degp_ref, htA_ref, htB_ref, dinv_ref):
    deg = degp_ref[0, :, 0:1] + 1.0
    dinv = lax.rsqrt(deg)
    h = jnp.dot(x_ref[...], wm_ref[...], preferred_element_type=jnp.float32)
    ht = jnp.maximum(h + bm_ref[...], 0.0) * dinv
    htA_ref[...] = ht[:, :64]
    htB_ref[...] = ht[:, 64:]
    dinv_ref[...] = jnp.broadcast_to(dinv, (dinv.shape[0], 8))


def _mid_body(acc_ref, htA_ref, htB_ref, dinv_ref, w1_ref, b1_ref, w2_ref,
              gA_ref, gB_ref):
    dinv = dinv_ref[:, 0:1]
    u = jnp.concatenate([acc_ref[0] + htA_ref[...], acc_ref[1] + htB_ref[...]],
                        axis=1) * dinv
    h1 = jnp.dot(u, w1_ref[...], preferred_element_type=jnp.float32)
    h1 = jnp.maximum(h1 + b1_ref[...], 0.0)
    g = jnp.dot(h1 * dinv, w2_ref[...], preferred_element_type=jnp.float32)
    gA_ref[...] = g[:, :32]
    gB_ref[...] = g[:, 32:]


def _fin_body(acc_ref, gA_ref, gB_ref, dinv_ref, b2_ref, out_ref):
    dinv = dinv_ref[:, 0:1]
    out = jnp.concatenate([acc_ref[0] + gA_ref[...], acc_ref[1] + gB_ref[...]],
                          axis=1)
    out_ref[...] = out * dinv + b2_ref[...]


def _tc_mlp(x, Wm, bm2, degp):
    return pl.pallas_call(
        _mlp_body,
        grid=(N // BLK,),
        in_specs=[
            pl.BlockSpec((BLK, 128), lambda i: (i, 0)),
            pl.BlockSpec((128, 128), lambda i: (0, 0)),
            pl.BlockSpec((1, 128), lambda i: (0, 0)),
            pl.BlockSpec((NC, BLK, 16), lambda i: (0, i, 0)),
        ],
        out_specs=[
            pl.BlockSpec((BLK, 64), lambda i: (i, 0)),
            pl.BlockSpec((BLK, 64), lambda i: (i, 0)),
            pl.BlockSpec((BLK, 8), lambda i: (i, 0)),
        ],
        out_shape=[
            jax.ShapeDtypeStruct((N, 64), jnp.float32),
            jax.ShapeDtypeStruct((N, 64), jnp.float32),
            jax.ShapeDtypeStruct((N, 8), jnp.float32),
        ],
    )(x, Wm, bm2, degp)


def _tc_mid(acc1, htA, htB, dinv, W1, b12, W2):
    return pl.pallas_call(
        _mid_body,
        grid=(N // BLK,),
        in_specs=[
            pl.BlockSpec((NC, BLK, 64), lambda i: (0, i, 0)),
            pl.BlockSpec((BLK, 64), lambda i: (i, 0)),
            pl.BlockSpec((BLK, 64), lambda i: (i, 0)),
            pl.BlockSpec((BLK, 8), lambda i: (i, 0)),
            pl.BlockSpec((128, 128), lambda i: (0, 0)),
            pl.BlockSpec((1, 128), lambda i: (0, 0)),
            pl.BlockSpec((128, 64), lambda i: (0, 0)),
        ],
        out_specs=[
            pl.BlockSpec((BLK, 32), lambda i: (i, 0)),
            pl.BlockSpec((BLK, 32), lambda i: (i, 0)),
        ],
        out_shape=[
            jax.ShapeDtypeStruct((N, 32), jnp.float32),
            jax.ShapeDtypeStruct((N, 32), jnp.float32),
        ],
    )(acc1, htA, htB, dinv, W1, b12, W2)


def _tc_fin(acc2, gA, gB, dinv, b22):
    return pl.pallas_call(
        _fin_body,
        grid=(N // BLK,),
        in_specs=[
            pl.BlockSpec((NC, BLK, 32), lambda i: (0, i, 0)),
            pl.BlockSpec((BLK, 32), lambda i: (i, 0)),
            pl.BlockSpec((BLK, 32), lambda i: (i, 0)),
            pl.BlockSpec((BLK, 8), lambda i: (i, 0)),
            pl.BlockSpec((1, 64), lambda i: (0, 0)),
        ],
        out_specs=pl.BlockSpec((BLK, 64), lambda i: (i, 0)),
        out_shape=jax.ShapeDtypeStruct((N, 64), jnp.float32),
    )(acc2, gA, gB, dinv, b22)


def kernel(x, edge_index, Wm, bm, W1, b1, W2, b2):
    src2 = edge_index[0].reshape(NS, CH, K)
    dst2 = edge_index[1].reshape(NS, CH, K)

    degp = _make_deg()(dst2)
    htA, htB, dinv = _tc_mlp(x, Wm, bm.reshape(1, -1), degp)
    acc1 = _make_prop(64)(htA, htB, src2, dst2)
    gA, gB = _tc_mid(acc1, htA, htB, dinv, W1, b1.reshape(1, -1), W2)
    acc2 = _make_prop(32)(gA, gB, src2, dst2)
    return _tc_fin(acc2, gA, gB, dinv, b2.reshape(1, -1))


# speedup vs baseline: 37.4907x; 1.5642x over previous
import functools

import jax
import jax.numpy as jnp
from jax import lax
from jax.experimental import pallas as pl
from jax.experimental.pallas import tpu as pltpu
from jax.experimental.pallas import tpu_sc as plsc

N = 10000
N_PAD = 10240
E = 320000
NC = 2
NS = 16
K = 80
CHT = E // (NS * K)
CHW = CHT // NC
NPT = N_PAD // NS
ZR = 128
RB = 5

_mesh = lambda: plsc.VectorSubcoreMesh(core_axis_name="c", subcore_axis_name="s")
_sc_params = lambda: pltpu.CompilerParams(use_tc_tiling_on_sc=False)


def _zero_fill(ref, nrows, width):
    z = jnp.zeros((16,), jnp.float32)

    def row(r, _):
        def col(cc, __):
            ref[r, pl.ds(cc * 16, 16)] = z
            return 0

        lax.fori_loop(0, width // 16, col, 0)
        return 0

    lax.fori_loop(0, nrows, row, 0)


def _zero_acc(zbuf, acc, s):
    for t in range(NPT // ZR):
        pltpu.sync_copy(zbuf, acc.at[pl.ds(s * NPT + t * ZR, ZR)])


@functools.lru_cache(maxsize=None)
def _make_deg():
    @functools.partial(
        pl.kernel,
        out_type=jax.ShapeDtypeStruct((NC, N_PAD, 16), jnp.float32),
        mesh=_mesh(),
        compiler_params=_sc_params(),
        scratch_types=[
            pltpu.VMEM((CHW, K), jnp.int32),
            pltpu.VMEM((K, 16), jnp.float32),
            pltpu.VMEM((ZR, 16), jnp.float32),
            pltpu.VMEM_SHARED((N_PAD, 16), jnp.float32),
            pltpu.SemaphoreType.DMA,
        ],
    )
    def deg_kernel(dst_hbm, out_hbm, dstv, ones_v, zbuf, acc, ssem):
        c = lax.axis_index("c")
        s = lax.axis_index("s")

        _zero_fill(zbuf, ZR, 16)
        one = jnp.ones((16,), jnp.float32)

        def fill_ones(r, _):
            ones_v[r, :] = one
            return 0

        lax.fori_loop(0, K, fill_ones, 0)
        _zero_acc(zbuf, acc, s)
        pltpu.sync_copy(dst_hbm.at[s, pl.ds(c * CHW, CHW)], dstv)
        plsc.subcore_barrier()

        def group(g, _):
            for t in range(RB):
                pltpu.async_copy(ones_v, acc.at[dstv.at[g * RB + t]], ssem,
                                 add=True)
            for t in range(RB):
                pltpu.make_async_copy(ones_v, acc.at[dstv.at[0]], ssem).wait()
            return 0

        lax.fori_loop(0, CHW // RB, group, 0)
        plsc.subcore_barrier()
        pltpu.sync_copy(acc.at[pl.ds(s * NPT, NPT)], out_hbm.at[c, pl.ds(s * NPT, NPT)])

    return deg_kernel


def _edge_pipeline(tab, srcv, dstv, rows, gsems, ssems, acc, n_chunks):
    M = n_chunks // RB

    def gissue(j, b):
        pltpu.async_copy(tab.at[srcv.at[j]], rows[b], gsems[b])

    def gwait(b):
        pltpu.make_async_copy(tab.at[srcv.at[0]], rows[b], gsems[b]).wait()

    def sissue(j, b):
        pltpu.async_copy(rows[b], acc.at[dstv.at[j]], ssems[b], add=True)

    def swait(b):
        pltpu.make_async_copy(rows[b], acc.at[dstv.at[0]], ssems[b]).wait()

    for b in range(3):
        gissue(b, b)
    for b in range(RB):
        b2 = (b - 2) % RB
        if b >= 2:
            swait(b2)
        gissue(b + 3, b2)
        gwait(b)
        sissue(b, b)

    def step(m, _):
        for b in range(RB):
            j = m * RB + b
            b2 = (b - 2) % RB
            swait(b2)
            gissue(j + 3, b2)
            gwait(b)
            sissue(j, b)
        return 0

    lax.fori_loop(1, M - 1, step, 0)

    base = (M - 1) * RB
    for b in range(RB):
        if b < 2:
            b2 = (b - 2) % RB
            swait(b2)
            gissue(base + b + 3, b2)
        gwait(b)
        sissue(base + b, b)
    for b in range(RB):
        swait(b)


def _prop_scratch(DH, n_chunks):
    return [
        pltpu.VMEM((n_chunks, K), jnp.int32),
        pltpu.VMEM((n_chunks, K), jnp.int32),
    ] + [pltpu.VMEM((K, DH), jnp.float32) for _ in range(RB)] + [
        pltpu.VMEM((ZR, DH), jnp.float32),
        pltpu.VMEM_SHARED((N_PAD, DH), jnp.float32),
    ] + [pltpu.SemaphoreType.DMA for _ in range(2 * RB)]


@functools.lru_cache(maxsize=None)
def _make_prop_col(DH):
    @functools.partial(
        pl.kernel,
        out_type=jax.ShapeDtypeStruct((NC, N_PAD, DH), jnp.float32),
        mesh=_mesh(),
        compiler_params=_sc_params(),
        scratch_types=_prop_scratch(DH, CHT),
    )
    def prop_kernel(tabA_hbm, tabB_hbm, src_hbm, dst_hbm, out_hbm,
                    srcv, dstv, r0, r1, r2, r3, r4, zbuf, acc,
                    g0, g1, g2, g3, g4, s0, s1, s2, s3, s4):
        c = lax.axis_index("c")
        s = lax.axis_index("s")
        rows = [r0, r1, r2, r3, r4]
        gsems = [g0, g1, g2, g3, g4]
        ssems = [s0, s1, s2, s3, s4]

        _zero_fill(zbuf, ZR, DH)
        _zero_acc(zbuf, acc, s)
        pltpu.sync_copy(src_hbm.at[s], srcv)
        pltpu.sync_copy(dst_hbm.at[s], dstv)
        plsc.subcore_barrier()

        @pl.when(c == 0)
        def _():
            _edge_pipeline(tabA_hbm, srcv, dstv, rows, gsems, ssems, acc, CHT)

        @pl.when(c == 1)
        def _():
            _edge_pipeline(tabB_hbm, srcv, dstv, rows, gsems, ssems, acc, CHT)

        plsc.subcore_barrier()
        pltpu.sync_copy(acc.at[pl.ds(s * NPT, NPT)], out_hbm.at[c, pl.ds(s * NPT, NPT)])

    return prop_kernel


@functools.lru_cache(maxsize=None)
def _make_prop_edge(DH):
    @functools.partial(
        pl.kernel,
        out_type=jax.ShapeDtypeStruct((NC, N_PAD, DH), jnp.float32),
        mesh=_mesh(),
        compiler_params=_sc_params(),
        scratch_types=_prop_scratch(DH, CHW),
    )
    def prop_kernel(tab_hbm, src_hbm, dst_hbm, out_hbm,
                    srcv, dstv, r0, r1, r2, r3, r4, zbuf, acc,
                    g0, g1, g2, g3, g4, s0, s1, s2, s3, s4):
        c = lax.axis_index("c")
        s = lax.axis_index("s")
        rows = [r0, r1, r2, r3, r4]
        gsems = [g0, g1, g2, g3, g4]
        ssems = [s0, s1, s2, s3, s4]

        _zero_fill(zbuf, ZR, DH)
        _zero_acc(zbuf, acc, s)
        pltpu.sync_copy(src_hbm.at[s, pl.ds(c * CHW, CHW)], srcv)
        pltpu.sync_copy(dst_hbm.at[s, pl.ds(c * CHW, CHW)], dstv)
        plsc.subcore_barrier()

        _edge_pipeline(tab_hbm, srcv, dstv, rows, gsems, ssems, acc, CHW)

        plsc.subcore_barrier()
        pltpu.sync_copy(acc.at[pl.ds(s * NPT, NPT)], out_hbm.at[c, pl.ds(s * NPT, NPT)])

    return prop_kernel


BLK = 2000


def _mlp_body(x_ref, wm_ref, bm_ref, degp_ref, htA_ref, htB_ref, dinv_ref):
    deg = degp_ref[0, :, 0:1] + degp_ref[1, :, 0:1] + 1.0
    dinv = lax.rsqrt(deg)
    h = jnp.dot(x_ref[...], wm_ref[...], preferred_element_type=jnp.float32)
    ht = jnp.maximum(h + bm_ref[...], 0.0) * dinv
    htA_ref[...] = ht[:, :64]
    htB_ref[...] = ht[:, 64:]
    dinv_ref[...] = jnp.broadcast_to(dinv, (dinv.shape[0], 8))


def _mid_body(acc_ref, htA_ref, htB_ref, dinv_ref, w1_ref, b1_ref, w2_ref, g_ref):
    dinv = dinv_ref[:, 0:1]
    u = jnp.concatenate([acc_ref[0] + htA_ref[...], acc_ref[1] + htB_ref[...]],
                        axis=1) * dinv
    h1 = jnp.dot(u, w1_ref[...], preferred_element_type=jnp.float32)
    h1 = jnp.maximum(h1 + b1_ref[...], 0.0)
    g_ref[...] = jnp.dot(h1 * dinv, w2_ref[...], preferred_element_type=jnp.float32)


def _fin_body(acc_ref, g_ref, dinv_ref, b2_ref, out_ref):
    dinv = dinv_ref[:, 0:1]
    out_ref[...] = (acc_ref[0] + acc_ref[1] + g_ref[...]) * dinv + b2_ref[...]


def _tc_mlp(x, Wm, bm2, degp):
    return pl.pallas_call(
        _mlp_body,
        grid=(N // BLK,),
        in_specs=[
            pl.BlockSpec((BLK, 128), lambda i: (i, 0)),
            pl.BlockSpec((128, 128), lambda i: (0, 0)),
            pl.BlockSpec((1, 128), lambda i: (0, 0)),
            pl.BlockSpec((NC, BLK, 16), lambda i: (0, i, 0)),
        ],
        out_specs=[
            pl.BlockSpec((BLK, 64), lambda i: (i, 0)),
            pl.BlockSpec((BLK, 64), lambda i: (i, 0)),
            pl.BlockSpec((BLK, 8), lambda i: (i, 0)),
        ],
        out_shape=[
            jax.ShapeDtypeStruct((N, 64), jnp.float32),
            jax.ShapeDtypeStruct((N, 64), jnp.float32),
            jax.ShapeDtypeStruct((N, 8), jnp.float32),
        ],
    )(x, Wm, bm2, degp)


def _tc_mid(acc1, htA, htB, dinv, W1, b12, W2):
    return pl.pallas_call(
        _mid_body,
        grid=(N // BLK,),
        in_specs=[
            pl.BlockSpec((NC, BLK, 64), lambda i: (0, i, 0)),
            pl.BlockSpec((BLK, 64), lambda i: (i, 0)),
            pl.BlockSpec((BLK, 64), lambda i: (i, 0)),
            pl.BlockSpec((BLK, 8), lambda i: (i, 0)),
            pl.BlockSpec((128, 128), lambda i: (0, 0)),
            pl.BlockSpec((1, 128), lambda i: (0, 0)),
            pl.BlockSpec((128, 64), lambda i: (0, 0)),
        ],
        out_specs=pl.BlockSpec((BLK, 64), lambda i: (i, 0)),
        out_shape=jax.ShapeDtypeStruct((N, 64), jnp.float32),
    )(acc1, htA, htB, dinv, W1, b12, W2)


def _tc_fin(acc2, g, dinv, b22):
    return pl.pallas_call(
        _fin_body,
        grid=(N // BLK,),
        in_specs=[
            pl.BlockSpec((NC, BLK, 64), lambda i: (0, i, 0)),
            pl.BlockSpec((BLK, 64), lambda i: (i, 0)),
            pl.BlockSpec((BLK, 8), lambda i: (i, 0)),
            pl.BlockSpec((1, 64), lambda i: (0, 0)),
        ],
        out_specs=pl.BlockSpec((BLK, 64), lambda i: (i, 0)),
        out_shape=jax.ShapeDtypeStruct((N, 64), jnp.float32),
    )(acc2, g, dinv, b22)


def kernel(x, edge_index, Wm, bm, W1, b1, W2, b2):
    src2 = edge_index[0].reshape(NS, CHT, K)
    dst2 = edge_index[1].reshape(NS, CHT, K)

    degp = _make_deg()(dst2)
    htA, htB, dinv = _tc_mlp(x, Wm, bm.reshape(1, -1), degp)
    acc1 = _make_prop_col(64)(htA, htB, src2, dst2)
    g = _tc_mid(acc1, htA, htB, dinv, W1, b1.reshape(1, -1), W2)
    acc2 = _make_prop_edge(64)(g, src2, dst2)
    return _tc_fin(acc2, g, dinv, b2.reshape(1, -1))
